# Initial kernel scaffold; baseline (speedup 1.0000x reference)
import jax
import jax.numpy as jnp
from jax.experimental import pallas as pl


def kernel(token_ids, table):
    raise NotImplementedError("write your pallas kernel here")


# speedup vs baseline: 1.5983x; 1.5983x over previous
import functools

import jax
import jax.numpy as jnp
from jax import lax
from jax.experimental import pallas as pl
from jax.experimental.pallas import tpu as pltpu
from jax.experimental.pallas import tpu_sc as plsc

_NBUF = 2
_CHUNK = 1600


@functools.partial(jax.jit, static_argnames=("n", "d"))
def _emb_lookup(idx, table, n, d):
    info = plsc.get_sparse_core_info()
    nc, ns = info.num_cores, info.num_subcores
    nw = nc * ns
    b_per_w = n // nw
    nch = b_per_w // _CHUNK

    mesh = plsc.VectorSubcoreMesh(core_axis_name="c", subcore_axis_name="s")

    @functools.partial(
        pl.kernel,
        mesh=mesh,
        compiler_params=pltpu.CompilerParams(use_tc_tiling_on_sc=False),
        out_type=jax.ShapeDtypeStruct((n, d), jnp.float32),
        scratch_types=[
            pltpu.VMEM((b_per_w,), jnp.int32),
            pltpu.VMEM((_NBUF, _CHUNK, d), jnp.float32),
            pltpu.SemaphoreType.DMA,
            pltpu.SemaphoreType.DMA,
        ],
    )
    def emb(idx_hbm, table_hbm, out_hbm, idx_v, rows_v, gsem, ssem):
        wid = lax.axis_index("s") * nc + lax.axis_index("c")
        base = wid * b_per_w
        pltpu.sync_copy(idx_hbm.at[pl.ds(base, b_per_w)], idx_v)

        gathers = [None] * _NBUF
        stores = [None] * _NBUF

        def start_gather(c):
            buf = c % _NBUF
            gathers[buf] = pltpu.async_copy(
                table_hbm.at[idx_v.at[pl.ds(c * _CHUNK, _CHUNK)]],
                rows_v.at[buf],
                gsem,
            )

        for c in range(min(_NBUF - 1, nch)):
            start_gather(c)
        for c in range(nch):
            buf = c % _NBUF
            if c > 0:
                stores[(c - 1) % _NBUF].wait()
            nxt = c + _NBUF - 1
            if nxt < nch:
                start_gather(nxt)
            gathers[buf].wait()
            stores[buf] = pltpu.async_copy(
                rows_v.at[buf],
                out_hbm.at[pl.ds(base + c * _CHUNK, _CHUNK)],
                ssem,
            )
        if nch > 0:
            stores[(nch - 1) % _NBUF].wait()

    return emb(idx, table)


def kernel(token_ids, table):
    b, s = token_ids.shape
    v, d = table.shape
    n = b * s
    idx = token_ids.reshape(n).astype(jnp.int32)
    out = _emb_lookup(idx, table, n, d)
    return out.reshape(b, s, d)
